# baseline (speedup 1.0000x reference)
import functools

import jax
import jax.numpy as jnp
from jax import lax
from jax.experimental import pallas as pl
from jax.experimental.pallas import tpu as pltpu
from jax.experimental.pallas import tpu_sc as plsc

BATCH = 16384
D = 128
NC = 2
NS = 16
NW = NC * NS
P = 4
PB = BATCH // P
CHUNK = PB // NW


def _gather_body(user_table, item_table, uidx_hbm, iidx_hbm, u_out, v_out,
                 uidx_v, iidx_v, urows_v, irows_v, sem):
    wid = lax.axis_index("s") * NC + lax.axis_index("c")
    base = wid * CHUNK
    pltpu.sync_copy(uidx_hbm.at[wid], uidx_v)
    pltpu.sync_copy(iidx_hbm.at[wid], iidx_v)
    ucp = pltpu.async_copy(user_table.at[uidx_v], urows_v, sem)
    icp = pltpu.async_copy(item_table.at[iidx_v], irows_v, sem)
    ucp.wait()
    pltpu.sync_copy(urows_v, u_out.at[pl.ds(base, CHUNK)])
    icp.wait()
    pltpu.sync_copy(irows_v, v_out.at[pl.ds(base, CHUNK)])


@functools.lru_cache(maxsize=1)
def _make_gather():
    mesh = plsc.VectorSubcoreMesh(core_axis_name="c", subcore_axis_name="s")
    return functools.partial(
        pl.kernel,
        mesh=mesh,
        out_type=[
            jax.ShapeDtypeStruct((PB, D), jnp.float32),
            jax.ShapeDtypeStruct((PB, D), jnp.float32),
        ],
        scratch_types=[
            pltpu.VMEM((CHUNK,), jnp.int32),
            pltpu.VMEM((CHUNK,), jnp.int32),
            pltpu.VMEM((CHUNK, D), jnp.float32),
            pltpu.VMEM((CHUNK, D), jnp.float32),
            pltpu.SemaphoreType.DMA,
        ],
    )(_gather_body)


BLK = 1024


def _tower(x, w1, b1, w2, b2):
    h = lax.dot_general(x, w1, (((1,), (1,)), ((), ())),
                        preferred_element_type=jnp.float32)
    h = jnp.maximum(h + b1, 0.0)
    y = lax.dot_general(h, w2, (((1,), (1,)), ((), ())),
                        preferred_element_type=jnp.float32) + b2
    n = jnp.sqrt(jnp.sum(y * y, axis=1, keepdims=True))
    return y / jnp.maximum(n, 1e-12)


def _mlp_body_first(uv, iv, w1u, b1u, w2u, b2u, w1i, b1i, w2i, b2i,
                    u_out, v_out):
    u_out[...] = _tower(uv[...], w1u[...], b1u[...], w2u[...], b2u[...])
    v_out[...] = _tower(iv[...], w1i[...], b1i[...], w2i[...], b2i[...])


def _mlp_body_carry(uv, iv, w1u, b1u, w2u, b2u, w1i, b1i, w2i, b2i,
                    u_carry, v_carry, u_out, v_out):
    _mlp_body_first(uv, iv, w1u, b1u, w2u, b2u, w1i, b1i, w2i, b2i,
                    u_out, v_out)


def _mlp_phase(p, u_carry, v_carry, uv, iv, *weights):
    vec_spec = pl.BlockSpec((BLK, D), lambda i: (i, 0))
    w_spec = pl.BlockSpec((D, D), lambda i: (0, 0))
    b_spec = pl.BlockSpec((1, D), lambda i: (0, 0))
    out_spec = pl.BlockSpec((BLK, D), lambda i, p=p: (p * (PB // BLK) + i, 0))
    any_spec = pl.BlockSpec(memory_space=pl.ANY)
    out_shape = [
        jax.ShapeDtypeStruct((BATCH, D), jnp.float32),
        jax.ShapeDtypeStruct((BATCH, D), jnp.float32),
    ]
    in_specs = [vec_spec, vec_spec,
                w_spec, b_spec, w_spec, b_spec,
                w_spec, b_spec, w_spec, b_spec]
    if p == 0:
        return pl.pallas_call(
            _mlp_body_first,
            grid=(PB // BLK,),
            in_specs=in_specs,
            out_specs=[out_spec, out_spec],
            out_shape=out_shape,
        )(uv, iv, *weights)
    return pl.pallas_call(
        _mlp_body_carry,
        grid=(PB // BLK,),
        in_specs=in_specs + [any_spec, any_spec],
        out_specs=[out_spec, out_spec],
        out_shape=out_shape,
        input_output_aliases={10: 0, 11: 1},
    )(uv, iv, *weights, u_carry, v_carry)


def kernel(user_ids, item_ids, user_table, item_table,
           W1u, b1u, W2u, b2u, W1i, b1i, W2i, b2i):
    uidx = user_ids.astype(jnp.int32).reshape(P, NW, CHUNK)
    iidx = item_ids.astype(jnp.int32).reshape(P, NW, CHUNK)
    gather = _make_gather()
    weights = (W1u, b1u.reshape(1, D), W2u, b2u.reshape(1, D),
               W1i, b1i.reshape(1, D), W2i, b2i.reshape(1, D))
    pieces = [gather(user_table, item_table, uidx[p], iidx[p])
              for p in range(P)]
    u = v = None
    for p, (uv_p, vv_p) in enumerate(pieces):
        u, v = _mlp_phase(p, u, v, uv_p, vv_p, *weights)
    return (u, v)
